# baseline (speedup 1.0000x reference)
import functools

import jax
import jax.numpy as jnp
from jax import lax
from jax.experimental import pallas as pl
from jax.experimental.pallas import tpu as pltpu
from jax.experimental.pallas import tpu_sc as plsc

B = 8
NQ = 5000
NT = 1000
L = 16

PRED_CHUNK = (NQ + L - 1) // L
TGT_CHUNK = (NT + L - 1) // L
PAD_N = PRED_CHUNK * L

RADIX_BITS = 8
NBINS = 1 << RADIX_BITS
NPASS = 4
OUT_PAD = 1024


def _sortable_key(area):
    a = area + 0.0
    u = plsc.bitcast(a, jnp.int32)
    s = lax.shift_right_arithmetic(u, 31)
    m = u ^ (s | jnp.int32(-2147483648))
    return ~m


def _radix_argsort(boxes_v, ka, kb, va, vb, hist, inc_v, base_v, rank_v,
                   n, chunk):
    lane = lax.iota(jnp.int32, L)
    zeros = jnp.zeros((L,), jnp.int32)
    ones = jnp.ones((L,), jnp.int32)

    @plsc.parallel_loop(0, chunk, unroll=4)
    def _build(i):
        e = lane * chunk + i
        x1 = plsc.load_gather(boxes_v, [e, zeros])
        y1 = plsc.load_gather(boxes_v, [e, ones])
        x2 = plsc.load_gather(boxes_v, [e, ones + 1])
        y2 = plsc.load_gather(boxes_v, [e, ones + 2])
        k = _sortable_key((x2 - x1) * (y2 - y1))
        k = jnp.where(e < n, k, jnp.int32(-1))
        ka[pl.ds(i * L, L)] = k
        va[pl.ds(i * L, L)] = e

    src = (ka, va)
    dst = (kb, vb)
    for p in range(NPASS):
        shift = p * RADIX_BITS
        ks, vs = src
        kd, vd = dst

        @plsc.parallel_loop(0, NBINS, unroll=8)
        def _clear(j):
            hist[pl.ds(j * L, L)] = zeros

        def histo(i, _):
            k = ks[pl.ds(i * L, L)]
            d = lax.shift_right_logical(k, shift) & (NBINS - 1)
            h = d * L + lane
            rank_v[pl.ds(i * L, L)] = plsc.load_gather(hist, [h])
            plsc.addupdate_scatter(hist, [h], ones)
            return 0

        lax.fori_loop(0, chunk, histo, 0, unroll=4)

        @plsc.parallel_loop(0, NBINS, unroll=4)
        def _scan_a(j):
            inc_v[pl.ds(j * L, L)] = plsc.cumsum(hist[pl.ds(j * L, L)])

        def scan_b(t, carry):
            tot = plsc.load_gather(inc_v, [(t * L + lane) * L + 15])
            c2 = plsc.cumsum(tot)
            base_v[pl.ds(t * L, L)] = c2 - tot + carry
            return carry + jnp.sum(tot)

        lax.fori_loop(0, L, scan_b, jnp.int32(0))

        last_pass = p == NPASS - 1

        @plsc.parallel_loop(0, chunk, unroll=4)
        def _permute(i):
            k = ks[pl.ds(i * L, L)]
            v = vs[pl.ds(i * L, L)]
            r = rank_v[pl.ds(i * L, L)]
            d = lax.shift_right_logical(k, shift) & (NBINS - 1)
            h = d * L + lane
            inc = plsc.load_gather(inc_v, [h])
            cnt = plsc.load_gather(hist, [h])
            b = plsc.load_gather(base_v, [d])
            o = b + inc - cnt + r
            if last_pass:
                phys = o
            else:
                magic = (1 << 23) // chunk + 1
                q = lax.shift_right_logical(o * magic, 23)
                phys = (o - q * chunk) * L + q
            plsc.store_scatter(kd, [phys], k)
            plsc.store_scatter(vd, [phys], v)

        src, dst = dst, src


def _matcher_body(pred_hbm, tgt_hbm, out_pred, out_tgt,
                  boxes_v, ka, kb, va, vb, hist, inc_v, base_v, rank_v):
    c = lax.axis_index("c")
    s = lax.axis_index("s")
    is_pred = s < 4
    is_tgt = (s >= 4) & (s < 8)
    pred_row = c * 4 + s
    tgt_row = c * 4 + (s - 4)

    @pl.when(is_pred)
    def _():
        pltpu.sync_copy(pred_hbm.at[pred_row], boxes_v.at[pl.ds(0, NQ)])
        _radix_argsort(boxes_v, ka, kb, va, vb, hist, inc_v, base_v, rank_v,
                       NQ, PRED_CHUNK)
        pltpu.sync_copy(va.at[pl.ds(0, NT)], out_pred.at[pred_row])

    @pl.when(is_tgt)
    def _():
        pltpu.sync_copy(tgt_hbm.at[tgt_row], boxes_v.at[pl.ds(0, NT)])
        _radix_argsort(boxes_v, ka, kb, va, vb, hist, inc_v, base_v, rank_v,
                       NT, TGT_CHUNK)
        pltpu.sync_copy(va.at[pl.ds(0, NT)], out_tgt.at[tgt_row])


@jax.jit
def _match(pred_boxes, target_boxes):
    run = functools.partial(
        pl.kernel,
        out_type=[
            jax.ShapeDtypeStruct((B, NT), jnp.int32),
            jax.ShapeDtypeStruct((B, NT), jnp.int32),
        ],
        mesh=plsc.VectorSubcoreMesh(core_axis_name="c", subcore_axis_name="s"),
        compiler_params=pltpu.CompilerParams(
            needs_layout_passes=False, use_tc_tiling_on_sc=False),
        scratch_types=[
            pltpu.VMEM((PAD_N, 4), jnp.float32),
            pltpu.VMEM((PAD_N,), jnp.int32),
            pltpu.VMEM((PAD_N,), jnp.int32),
            pltpu.VMEM((PAD_N,), jnp.int32),
            pltpu.VMEM((PAD_N,), jnp.int32),
            pltpu.VMEM((NBINS * L,), jnp.int32),
            pltpu.VMEM((NBINS * L,), jnp.int32),
            pltpu.VMEM((NBINS,), jnp.int32),
            pltpu.VMEM((PAD_N,), jnp.int32),
        ],
    )(_matcher_body)
    return run(pred_boxes, target_boxes)


def kernel(logits, pred_boxes, target_boxes, class_labels):
    del logits, class_labels
    matched_pred, matched_tgt = _match(pred_boxes, target_boxes)
    return (matched_pred, matched_tgt)


# speedup vs baseline: 1.4935x; 1.4935x over previous
import functools

import jax
import jax.numpy as jnp
from jax import lax
from jax.experimental import pallas as pl
from jax.experimental.pallas import tpu as pltpu
from jax.experimental.pallas import tpu_sc as plsc

B = 8
NQ = 5000
NT = 1000
L = 16

PRED_CHUNK = (NQ + L - 1) // L
TGT_CHUNK = (NT + L - 1) // L
PAD_N = PRED_CHUNK * L

RADIX_BITS = 8
NBINS = 1 << RADIX_BITS
NPASS = 4
OUT_PAD = 1024


def _sortable_key(area):
    a = area + 0.0
    u = plsc.bitcast(a, jnp.int32)
    s = lax.shift_right_arithmetic(u, 31)
    m = u ^ (s | jnp.int32(-2147483648))
    return ~m


def _radix_argsort(boxes_v, ka, kb, va, vb, hist, inc_v, base_v, rank_v,
                   n, chunk):
    lane = lax.iota(jnp.int32, L)
    zeros = jnp.zeros((L,), jnp.int32)
    ones = jnp.ones((L,), jnp.int32)

    @plsc.parallel_loop(0, chunk, unroll=4)
    def _build(i):
        e = lane * chunk + i
        b4 = e * 4
        x1 = plsc.load_gather(boxes_v, [b4])
        y1 = plsc.load_gather(boxes_v, [b4 + 1])
        x2 = plsc.load_gather(boxes_v, [b4 + 2])
        y2 = plsc.load_gather(boxes_v, [b4 + 3])
        k = _sortable_key((x2 - x1) * (y2 - y1))
        k = jnp.where(e < n, k, jnp.int32(-1))
        ka[pl.ds(i * L, L)] = k
        va[pl.ds(i * L, L)] = e

    src = (ka, va)
    dst = (kb, vb)
    for p in range(NPASS):
        shift = p * RADIX_BITS
        ks, vs = src
        kd, vd = dst

        @plsc.parallel_loop(0, NBINS, unroll=8)
        def _clear(j):
            hist[pl.ds(j * L, L)] = zeros

        def histo(i, _):
            k = ks[pl.ds(i * L, L)]
            d = lax.shift_right_logical(k, shift) & (NBINS - 1)
            h = d * L + lane
            rank_v[pl.ds(i * L, L)] = plsc.load_gather(hist, [h])
            plsc.addupdate_scatter(hist, [h], ones)
            return 0

        lax.fori_loop(0, chunk, histo, 0, unroll=4)

        @plsc.parallel_loop(0, NBINS, unroll=4)
        def _scan_a(j):
            inc_v[pl.ds(j * L, L)] = plsc.cumsum(hist[pl.ds(j * L, L)])

        def scan_b(t, carry):
            tot = plsc.load_gather(inc_v, [(t * L + lane) * L + 15])
            c2 = plsc.cumsum(tot)
            base_v[pl.ds(t * L, L)] = c2 - tot + carry
            return carry + jnp.sum(tot)

        lax.fori_loop(0, L, scan_b, jnp.int32(0))

        last_pass = p == NPASS - 1

        @plsc.parallel_loop(0, chunk, unroll=4)
        def _permute(i):
            k = ks[pl.ds(i * L, L)]
            v = vs[pl.ds(i * L, L)]
            r = rank_v[pl.ds(i * L, L)]
            d = lax.shift_right_logical(k, shift) & (NBINS - 1)
            h = d * L + lane
            inc = plsc.load_gather(inc_v, [h])
            cnt = plsc.load_gather(hist, [h])
            b = plsc.load_gather(base_v, [d])
            o = b + inc - cnt + r
            if last_pass:
                phys = o
            else:
                magic = (1 << 23) // chunk + 1
                q = lax.shift_right_logical(o * magic, 23)
                phys = (o - q * chunk) * L + q
            plsc.store_scatter(kd, [phys], k)
            plsc.store_scatter(vd, [phys], v)

        src, dst = dst, src


def _matcher_body(pred_hbm, tgt_hbm, out_pred, out_tgt,
                  boxes_v, ka, kb, va, vb, hist, inc_v, base_v, rank_v):
    c = lax.axis_index("c")
    s = lax.axis_index("s")
    is_pred = s < 4
    is_tgt = (s >= 4) & (s < 8)
    pred_row = c * 4 + s
    tgt_row = c * 4 + (s - 4)

    @pl.when(is_pred)
    def _():
        pltpu.sync_copy(pred_hbm.at[pred_row], boxes_v.at[pl.ds(0, NQ * 4)])
        _radix_argsort(boxes_v, ka, kb, va, vb, hist, inc_v, base_v, rank_v,
                       NQ, PRED_CHUNK)
        pltpu.sync_copy(va.at[pl.ds(0, NT)],
                        out_pred.at[pl.ds(pred_row * OUT_PAD, NT)])

    @pl.when(is_tgt)
    def _():
        pltpu.sync_copy(tgt_hbm.at[tgt_row], boxes_v.at[pl.ds(0, NT * 4)])
        _radix_argsort(boxes_v, ka, kb, va, vb, hist, inc_v, base_v, rank_v,
                       NT, TGT_CHUNK)
        pltpu.sync_copy(va.at[pl.ds(0, NT)],
                        out_tgt.at[pl.ds(tgt_row * OUT_PAD, NT)])


@jax.jit
def _match(pred_boxes, target_boxes):
    run = functools.partial(
        pl.kernel,
        out_type=[
            jax.ShapeDtypeStruct((B * OUT_PAD,), jnp.int32),
            jax.ShapeDtypeStruct((B * OUT_PAD,), jnp.int32),
        ],
        mesh=plsc.VectorSubcoreMesh(core_axis_name="c", subcore_axis_name="s"),
        compiler_params=pltpu.CompilerParams(
            needs_layout_passes=False, use_tc_tiling_on_sc=False),
        scratch_types=[
            pltpu.VMEM((PAD_N * 4,), jnp.float32),
            pltpu.VMEM((PAD_N,), jnp.int32),
            pltpu.VMEM((PAD_N,), jnp.int32),
            pltpu.VMEM((PAD_N,), jnp.int32),
            pltpu.VMEM((PAD_N,), jnp.int32),
            pltpu.VMEM((NBINS * L,), jnp.int32),
            pltpu.VMEM((NBINS * L,), jnp.int32),
            pltpu.VMEM((NBINS,), jnp.int32),
            pltpu.VMEM((PAD_N,), jnp.int32),
        ],
    )(_matcher_body)
    mp, mt = run(pred_boxes.reshape(B, NQ * 4), target_boxes.reshape(B, NT * 4))
    return mp.reshape(B, OUT_PAD)[:, :NT], mt.reshape(B, OUT_PAD)[:, :NT]


def kernel(logits, pred_boxes, target_boxes, class_labels):
    del logits, class_labels
    matched_pred, matched_tgt = _match(pred_boxes, target_boxes)
    return (matched_pred, matched_tgt)


# speedup vs baseline: 1.4967x; 1.0021x over previous
import functools

import jax
import jax.numpy as jnp
from jax import lax
from jax.experimental import pallas as pl
from jax.experimental.pallas import tpu as pltpu
from jax.experimental.pallas import tpu_sc as plsc

B = 8
NQ = 5000
NT = 1000
L = 16

PRED_CHUNK = (NQ + L - 1) // L
TGT_CHUNK = (NT + L - 1) // L
PAD_N = PRED_CHUNK * L

RADIX_BITS = 8
NBINS = 1 << RADIX_BITS
NPASS = 4
OUT_PAD = 1024


def _sortable_key(area):
    a = area + 0.0
    u = plsc.bitcast(a, jnp.int32)
    s = lax.shift_right_arithmetic(u, 31)
    m = u ^ (s | jnp.int32(-2147483648))
    return ~m


def _radix_argsort(boxes_v, ka, kb, va, vb, hist, inc_v, base_v, rank_v,
                   n, chunk):
    lane = lax.iota(jnp.int32, L)
    zeros = jnp.zeros((L,), jnp.int32)
    ones = jnp.ones((L,), jnp.int32)

    @plsc.parallel_loop(0, chunk, unroll=4)
    def _build(i):
        e = lane * chunk + i
        b4 = e * 4
        x1 = plsc.load_gather(boxes_v, [b4])
        y1 = plsc.load_gather(boxes_v, [b4 + 1])
        x2 = plsc.load_gather(boxes_v, [b4 + 2])
        y2 = plsc.load_gather(boxes_v, [b4 + 3])
        k = _sortable_key((x2 - x1) * (y2 - y1))
        k = jnp.where(e < n, k, jnp.int32(-1))
        ka[pl.ds(i * L, L)] = k
        va[pl.ds(i * L, L)] = e

    src = (ka, va)
    dst = (kb, vb)
    for p in range(NPASS):
        shift = p * RADIX_BITS
        ks, vs = src
        kd, vd = dst

        @plsc.parallel_loop(0, NBINS, unroll=8)
        def _clear(j):
            hist[pl.ds(j * L, L)] = zeros

        def histo(i, _):
            k = ks[pl.ds(i * L, L)]
            d = lax.shift_right_logical(k, shift) & (NBINS - 1)
            h = d * L + lane
            rank_v[pl.ds(i * L, L)] = plsc.load_gather(hist, [h])
            plsc.addupdate_scatter(hist, [h], ones)
            return 0

        lax.fori_loop(0, chunk, histo, 0, unroll=4)

        @plsc.parallel_loop(0, NBINS, unroll=4)
        def _scan_a(j):
            inc_v[pl.ds(j * L, L)] = plsc.cumsum(hist[pl.ds(j * L, L)])

        def scan_b(t, carry):
            tot = plsc.load_gather(inc_v, [(t * L + lane) * L + 15])
            c2 = plsc.cumsum(tot)
            base_v[pl.ds(t * L, L)] = c2 - tot + carry
            return carry + jnp.sum(tot)

        lax.fori_loop(0, L, scan_b, jnp.int32(0))

        last_pass = p == NPASS - 1

        @plsc.parallel_loop(0, chunk, unroll=4)
        def _permute(i):
            k = ks[pl.ds(i * L, L)]
            v = vs[pl.ds(i * L, L)]
            r = rank_v[pl.ds(i * L, L)]
            d = lax.shift_right_logical(k, shift) & (NBINS - 1)
            h = d * L + lane
            inc = plsc.load_gather(inc_v, [h])
            cnt = plsc.load_gather(hist, [h])
            b = plsc.load_gather(base_v, [d])
            o = b + inc - cnt + r
            if last_pass:
                phys = o
            else:
                magic = (1 << 23) // chunk + 1
                q = lax.shift_right_logical(o * magic, 23)
                phys = (o - q * chunk) * L + q
            plsc.store_scatter(kd, [phys], k)
            plsc.store_scatter(vd, [phys], v)

        src, dst = dst, src


def _matcher_body(pred_hbm, tgt_hbm, out_pred, out_tgt,
                  boxes_v, ka, kb, va, vb, hist, inc_v, base_v, rank_v):
    c = lax.axis_index("c")
    s = lax.axis_index("s")
    is_pred = s < 4
    is_tgt = (s >= 4) & (s < 8)
    pred_row = c * 4 + s
    tgt_row = c * 4 + (s - 4)

    @pl.when(is_pred)
    def _():
        pltpu.sync_copy(pred_hbm.at[pred_row], boxes_v.at[pl.ds(0, NQ * 4)])
        _radix_argsort(boxes_v, ka, kb, va, vb, hist, inc_v, base_v, rank_v,
                       NQ, PRED_CHUNK)
        pltpu.sync_copy(va.at[pl.ds(0, NT)],
                        out_pred.at[pl.ds(pred_row * OUT_PAD, NT)])

    @pl.when(is_tgt)
    def _():
        pltpu.sync_copy(tgt_hbm.at[tgt_row], boxes_v.at[pl.ds(0, NT * 4)])
        _radix_argsort(boxes_v, ka, kb, va, vb, hist, inc_v, base_v, rank_v,
                       NT, TGT_CHUNK)
        pltpu.sync_copy(va.at[pl.ds(0, NT)],
                        out_tgt.at[pl.ds(tgt_row * OUT_PAD, NT)])


@jax.jit
def _match(pred_boxes, target_boxes):
    run = functools.partial(
        pl.kernel,
        out_type=[
            jax.ShapeDtypeStruct((B * OUT_PAD,), jnp.int32),
            jax.ShapeDtypeStruct((B * OUT_PAD,), jnp.int32),
        ],
        mesh=plsc.VectorSubcoreMesh(core_axis_name="c", subcore_axis_name="s"),
        compiler_params=pltpu.CompilerParams(
            needs_layout_passes=False, use_tc_tiling_on_sc=False,
            disable_bounds_checks=True, disable_semaphore_checks=True,
            skip_device_barrier=True),
        scratch_types=[
            pltpu.VMEM((PAD_N * 4,), jnp.float32),
            pltpu.VMEM((PAD_N,), jnp.int32),
            pltpu.VMEM((PAD_N,), jnp.int32),
            pltpu.VMEM((PAD_N,), jnp.int32),
            pltpu.VMEM((PAD_N,), jnp.int32),
            pltpu.VMEM((NBINS * L,), jnp.int32),
            pltpu.VMEM((NBINS * L,), jnp.int32),
            pltpu.VMEM((NBINS,), jnp.int32),
            pltpu.VMEM((PAD_N,), jnp.int32),
        ],
    )(_matcher_body)
    mp, mt = run(pred_boxes.reshape(B, NQ * 4), target_boxes.reshape(B, NT * 4))
    return mp.reshape(B, OUT_PAD)[:, :NT], mt.reshape(B, OUT_PAD)[:, :NT]


def kernel(logits, pred_boxes, target_boxes, class_labels):
    del logits, class_labels
    matched_pred, matched_tgt = _match(pred_boxes, target_boxes)
    return (matched_pred, matched_tgt)
